# Initial kernel scaffold; baseline (speedup 1.0000x reference)
import jax
import jax.numpy as jnp
from jax.experimental import pallas as pl


def kernel(h, x, edge_index, edge_attr, We1, be1, We2, be2, Wv11, bv11, Wv12, bv12, Wv21, bv21, Wv22, bv22):
    raise NotImplementedError("write your pallas kernel here")


# speedup vs baseline: 2.2544x; 2.2544x over previous
import functools

import jax
import jax.numpy as jnp
from jax import lax
from jax.experimental import pallas as pl
from jax.experimental.pallas import tpu as pltpu
from jax.experimental.pallas import tpu_sc as plsc

_NC = 2
_NS = 16
_NW = _NC * _NS


def _mesh():
    return plsc.VectorSubcoreMesh(
        core_axis_name="c", subcore_axis_name="s",
        num_cores=_NC, num_subcores=_NS)


def _sc_gather(h, row, col, chunk=80):
    n, hdim = h.shape
    e = row.shape[0]
    per_w = e // _NW
    n_chunks = per_w // chunk

    @functools.partial(
        pl.kernel,
        out_type=(jax.ShapeDtypeStruct((e, hdim), h.dtype),
                  jax.ShapeDtypeStruct((e, hdim), h.dtype)),
        mesh=_mesh(),
        scratch_types=[
            pltpu.VMEM((chunk,), jnp.int32),
            pltpu.VMEM((chunk,), jnp.int32),
            pltpu.VMEM((chunk, hdim), h.dtype),
            pltpu.VMEM((chunk, hdim), h.dtype),
            pltpu.SemaphoreType.DMA,
            pltpu.SemaphoreType.DMA,
        ],
    )
    def k(h_hbm, row_hbm, col_hbm, hr_hbm, hc_hbm,
          idr, idc, bufr, bufc, sem_r, sem_c):
        wid = lax.axis_index("s") * _NC + lax.axis_index("c")
        base = wid * per_w

        def body(j, carry):
            off = base + j * chunk
            pltpu.sync_copy(row_hbm.at[pl.ds(off, chunk)], idr)
            pltpu.sync_copy(col_hbm.at[pl.ds(off, chunk)], idc)
            cr = pltpu.async_copy(h_hbm.at[idr], bufr, sem_r)
            cc = pltpu.async_copy(h_hbm.at[idc], bufc, sem_c)
            cr.wait()
            cc.wait()
            pltpu.sync_copy(bufr, hr_hbm.at[pl.ds(off, chunk)])
            pltpu.sync_copy(bufc, hc_hbm.at[pl.ds(off, chunk)])
            return carry

        lax.fori_loop(0, n_chunks, body, 0)

    return k(h, row, col)


def _sc_scatter(ef, row, n, chunk=80):
    e, hdim = ef.shape
    per_tile = e // _NS
    n_chunks = per_tile // chunk
    half = (-(-n // (2 * 8)) * 8)
    rows_per_tile = -(-(half + 1) // (_NS * 8)) * 8
    acc_rows = rows_per_tile * _NS
    lanes = hdim // 16

    @functools.partial(
        pl.kernel,
        out_type=(jax.ShapeDtypeStruct((_NC, acc_rows, hdim), jnp.float32),
                  jax.ShapeDtypeStruct((_NC, acc_rows, hdim), jnp.float32)),
        mesh=_mesh(),
        scratch_types=[
            pltpu.VMEM((chunk,), jnp.int32),
            pltpu.VMEM((chunk,), jnp.int32),
            pltpu.VMEM((chunk, hdim), jnp.float32),
            pltpu.VMEM((chunk, hdim), jnp.float32),
            pltpu.VMEM((rows_per_tile, hdim), jnp.float32),
            pltpu.VMEM_SHARED((acc_rows, hdim), jnp.float32),
            pltpu.SemaphoreType.DMA,
        ],
    )
    def k(ef_hbm, row_hbm, sums_hbm, cnts_hbm,
          idx, idx2, buf, ones, zeros, acc_sh, sem):
        cid = lax.axis_index("c")
        sid = lax.axis_index("s")
        base = sid * per_tile
        r0 = sid * rows_per_tile
        lo = cid * half

        def fill(i, carry):
            r = i // lanes
            q = i % lanes
            zeros[r, pl.ds(q * 16, 16)] = jnp.zeros((16,), jnp.float32)
            return carry

        lax.fori_loop(0, rows_per_tile * lanes, fill, 0)

        def fill1(i, carry):
            r = i // lanes
            q = i % lanes
            ones[r, pl.ds(q * 16, 16)] = jnp.ones((16,), jnp.float32)
            return carry

        lax.fori_loop(0, chunk * lanes, fill1, 0)

        def zero_acc():
            pltpu.sync_copy(zeros, acc_sh.at[pl.ds(r0, rows_per_tile)])

        def localize(j, carry):
            del j, carry
            for q in range(chunk // 16):
                v = idx[pl.ds(q * 16, 16)] - lo
                ok = (v >= 0) & (v < half)
                idx2[pl.ds(q * 16, 16)] = jnp.where(ok, v, half)

        zero_acc()
        plsc.subcore_barrier()

        def sbody(j, carry):
            off = base + j * chunk
            pltpu.sync_copy(row_hbm.at[pl.ds(off, chunk)], idx)
            localize(j, carry)
            pltpu.async_copy(ef_hbm.at[pl.ds(off, chunk)], buf, sem).wait()
            pltpu.sync_copy(buf, acc_sh.at[idx2], add=True)
            return carry

        lax.fori_loop(0, n_chunks, sbody, 0)
        plsc.subcore_barrier()
        pltpu.sync_copy(acc_sh.at[pl.ds(r0, rows_per_tile)],
                        sums_hbm.at[cid, pl.ds(r0, rows_per_tile)])
        plsc.subcore_barrier()

        zero_acc()
        plsc.subcore_barrier()

        def cbody(j, carry):
            off = base + j * chunk
            pltpu.sync_copy(row_hbm.at[pl.ds(off, chunk)], idx)
            localize(j, carry)
            pltpu.sync_copy(ones, acc_sh.at[idx2], add=True)
            return carry

        lax.fori_loop(0, n_chunks, cbody, 0)
        plsc.subcore_barrier()
        pltpu.sync_copy(acc_sh.at[pl.ds(r0, rows_per_tile)],
                        cnts_hbm.at[cid, pl.ds(r0, rows_per_tile)])

    return k(ef, row), half


def _silu(v):
    return v * jax.nn.sigmoid(v)


def _dot_t(a, b):
    return lax.dot_general(a, b, (((1,), (1,)), ((), ())),
                           preferred_element_type=jnp.float32)


def _tc_edge_mlp(hr, hc, ea, we1, be1, we2, be2, block=2000):
    e, hdim = hr.shape
    efdim = ea.shape[1]
    grid = e // block

    def body(hr_ref, hc_ref, ea_ref, w1_ref, b1_ref, w2_ref, b2_ref, out_ref):
        w1 = w1_ref[...]
        g = (_dot_t(hr_ref[...], w1[:, :hdim])
             + _dot_t(hc_ref[...], w1[:, hdim:2 * hdim])
             + _dot_t(ea_ref[...], w1[:, 2 * hdim:])
             + b1_ref[...])
        g = _silu(g)
        f = _dot_t(g, w2_ref[...]) + b2_ref[...]
        out_ref[...] = _silu(f)

    return pl.pallas_call(
        body,
        grid=(grid,),
        in_specs=[
            pl.BlockSpec((block, hdim), lambda i: (i, 0)),
            pl.BlockSpec((block, hdim), lambda i: (i, 0)),
            pl.BlockSpec((block, efdim), lambda i: (i, 0)),
            pl.BlockSpec((hdim, 2 * hdim + efdim), lambda i: (0, 0)),
            pl.BlockSpec((1, hdim), lambda i: (0, 0)),
            pl.BlockSpec((hdim, hdim), lambda i: (0, 0)),
            pl.BlockSpec((1, hdim), lambda i: (0, 0)),
        ],
        out_specs=pl.BlockSpec((block, hdim), lambda i: (i, 0)),
        out_shape=jax.ShapeDtypeStruct((e, hdim), jnp.float32),
    )(hr, hc, ea, we1, be1.reshape(1, -1), we2, be2.reshape(1, -1))


def _tc_final(sums2, cnts2, n, half,
              wv11, bv11, wv12, bv12, wv21, bv21, wv22, bv22):
    hi = n - half

    def body(p_ref, c_ref, w11_ref, b11_ref, w12_ref, b12_ref,
             w21_ref, b21_ref, w22_ref, b22_ref, out_ref):
        sums = jnp.concatenate([p_ref[0, :half], p_ref[1, :hi]], axis=0)
        cnt = jnp.concatenate([c_ref[0, :half, :1], c_ref[1, :hi, :1]],
                              axis=0)
        node = sums / jnp.maximum(cnt, 1.0)

        t1 = _silu(_dot_t(node, w11_ref[...]) + b11_ref[...])
        v1 = _dot_t(w12_ref[...], t1) + b12_ref[...]
        t2 = _silu(_dot_t(node, w21_ref[...]) + b21_ref[...])
        v2 = _dot_t(w22_ref[...], t2) + b22_ref[...]

        n1 = jnp.sqrt(jnp.sum(v1 * v1, axis=0, keepdims=True))
        v1n = v1 / jnp.maximum(n1, 1e-12)
        v2 = v2 - jnp.sum(v2 * v1n, axis=0, keepdims=True) * v1n
        n2 = jnp.sqrt(jnp.sum(v2 * v2, axis=0, keepdims=True))
        v2n = v2 / jnp.maximum(n2, 1e-12)
        v3 = jnp.concatenate([
            v1n[1:2] * v2n[2:3] - v1n[2:3] * v2n[1:2],
            v1n[2:3] * v2n[0:1] - v1n[0:1] * v2n[2:3],
            v1n[0:1] * v2n[1:2] - v1n[1:2] * v2n[0:1],
        ], axis=0)
        out_ref[...] = jnp.concatenate([
            v1n[0:1], v2n[0:1], v3[0:1],
            v1n[1:2], v2n[1:2], v3[1:2],
            v1n[2:3], v2n[2:3], v3[2:3],
        ], axis=0)

    return pl.pallas_call(
        body,
        out_shape=jax.ShapeDtypeStruct((9, n), jnp.float32),
    )(sums2, cnts2, wv11, bv11.reshape(1, -1), wv12, bv12.reshape(3, 1),
      wv21, bv21.reshape(1, -1), wv22, bv22.reshape(3, 1))


def kernel(h, x, edge_index, edge_attr, We1, be1, We2, be2,
           Wv11, bv11, Wv12, bv12, Wv21, bv21, Wv22, bv22):
    del x
    n = h.shape[0]
    row = edge_index[0]
    col = edge_index[1]
    hr, hc = _sc_gather(h, row, col)
    ef2 = _tc_edge_mlp(hr, hc, edge_attr, We1, be1, We2, be2)
    (sums2, cnts2), half = _sc_scatter(ef2, row, n)
    out9 = _tc_final(sums2, cnts2, n, half, Wv11, bv11, Wv12, bv12,
                     Wv21, bv21, Wv22, bv22)
    return out9.T.reshape(n, 3, 3)


# speedup vs baseline: 2.4369x; 1.0810x over previous
import functools

import jax
import jax.numpy as jnp
from jax import lax
from jax.experimental import pallas as pl
from jax.experimental.pallas import tpu as pltpu
from jax.experimental.pallas import tpu_sc as plsc

_NC = 2
_NS = 16
_NW = _NC * _NS


def _mesh():
    return plsc.VectorSubcoreMesh(
        core_axis_name="c", subcore_axis_name="s",
        num_cores=_NC, num_subcores=_NS)


def _sc_gather(h, row, col, chunk=80):
    n, hdim = h.shape
    e = row.shape[0]
    per_w = e // _NW
    n_chunks = per_w // chunk

    @functools.partial(
        pl.kernel,
        out_type=(jax.ShapeDtypeStruct((e, hdim), h.dtype),
                  jax.ShapeDtypeStruct((e, hdim), h.dtype)),
        mesh=_mesh(),
        scratch_types=[
            pltpu.VMEM((chunk,), jnp.int32),
            pltpu.VMEM((chunk,), jnp.int32),
            pltpu.VMEM((chunk, hdim), h.dtype),
            pltpu.VMEM((chunk, hdim), h.dtype),
            pltpu.SemaphoreType.DMA,
            pltpu.SemaphoreType.DMA,
        ],
    )
    def k(h_hbm, row_hbm, col_hbm, hr_hbm, hc_hbm,
          idr, idc, bufr, bufc, sem_r, sem_c):
        wid = lax.axis_index("s") * _NC + lax.axis_index("c")
        base = wid * per_w

        def body(j, carry):
            off = base + j * chunk
            pltpu.sync_copy(row_hbm.at[pl.ds(off, chunk)], idr)
            pltpu.sync_copy(col_hbm.at[pl.ds(off, chunk)], idc)
            cr = pltpu.async_copy(h_hbm.at[idr], bufr, sem_r)
            cc = pltpu.async_copy(h_hbm.at[idc], bufc, sem_c)
            cr.wait()
            cc.wait()
            pltpu.sync_copy(bufr, hr_hbm.at[pl.ds(off, chunk)])
            pltpu.sync_copy(bufc, hc_hbm.at[pl.ds(off, chunk)])
            return carry

        lax.fori_loop(0, n_chunks, body, 0)

    return k(h, row, col)


def _sc_scatter(ef, row, n, chunk=80):
    e, hdim = ef.shape
    per_tile = e // _NS
    n_chunks = per_tile // chunk
    half = (-(-n // (2 * 8)) * 8)
    rows_per_tile = -(-(half + 1) // (_NS * 8)) * 8
    acc_rows = rows_per_tile * _NS
    lanes = hdim // 16
    zrows = rows_per_tile // 8

    @functools.partial(
        pl.kernel,
        out_type=(jax.ShapeDtypeStruct((_NC, acc_rows, hdim), jnp.float32),
                  jax.ShapeDtypeStruct((_NC, acc_rows, hdim), jnp.float32)),
        mesh=_mesh(),
        scratch_types=[
            pltpu.VMEM((chunk,), jnp.int32),
            pltpu.VMEM((chunk,), jnp.int32),
            pltpu.VMEM((chunk, hdim), jnp.float32),
            pltpu.VMEM((chunk, hdim), jnp.float32),
            pltpu.VMEM((zrows, hdim), jnp.float32),
            pltpu.VMEM_SHARED((acc_rows, hdim), jnp.float32),
            pltpu.VMEM_SHARED((acc_rows, hdim), jnp.float32),
            pltpu.SemaphoreType.DMA,
        ],
    )
    def k(ef_hbm, row_hbm, sums_hbm, cnts_hbm,
          idx, idx2, buf, ones, zeros, acc_sh, acc1_sh, sem):
        cid = lax.axis_index("c")
        sid = lax.axis_index("s")
        base = sid * per_tile
        r0 = sid * rows_per_tile
        lo = cid * half

        def fill(i, carry):
            r = i // lanes
            q = i % lanes
            zeros[r, pl.ds(q * 16, 16)] = jnp.zeros((16,), jnp.float32)
            return carry

        lax.fori_loop(0, zrows * lanes, fill, 0)

        def fill1(i, carry):
            r = i // lanes
            q = i % lanes
            ones[r, pl.ds(q * 16, 16)] = jnp.ones((16,), jnp.float32)
            return carry

        lax.fori_loop(0, chunk * lanes, fill1, 0)

        def localize(j, carry):
            del j, carry
            for q in range(chunk // 16):
                v = idx[pl.ds(q * 16, 16)] - lo
                ok = (v >= 0) & (v < half)
                idx2[pl.ds(q * 16, 16)] = jnp.where(ok, v, half)

        for z in range(rows_per_tile // zrows):
            pltpu.sync_copy(zeros, acc_sh.at[pl.ds(r0 + z * zrows, zrows)])
            pltpu.sync_copy(zeros, acc1_sh.at[pl.ds(r0 + z * zrows, zrows)])
        plsc.subcore_barrier()

        def sbody(j, carry):
            off = base + j * chunk
            pltpu.sync_copy(row_hbm.at[pl.ds(off, chunk)], idx)
            localize(j, carry)
            pltpu.async_copy(ef_hbm.at[pl.ds(off, chunk)], buf, sem).wait()
            pltpu.sync_copy(buf, acc_sh.at[idx2], add=True)
            pltpu.sync_copy(ones, acc1_sh.at[idx2], add=True)
            return carry

        lax.fori_loop(0, n_chunks, sbody, 0)
        plsc.subcore_barrier()
        pltpu.sync_copy(acc_sh.at[pl.ds(r0, rows_per_tile)],
                        sums_hbm.at[cid, pl.ds(r0, rows_per_tile)])
        pltpu.sync_copy(acc1_sh.at[pl.ds(r0, rows_per_tile)],
                        cnts_hbm.at[cid, pl.ds(r0, rows_per_tile)])

    return k(ef, row), half


def _silu(v):
    return v * jax.nn.sigmoid(v)


def _dot_t(a, b):
    return lax.dot_general(a, b, (((1,), (1,)), ((), ())),
                           preferred_element_type=jnp.float32)


def _tc_edge_mlp(hr, hc, ea, we1, be1, we2, be2, block=2000):
    e, hdim = hr.shape
    efdim = ea.shape[1]
    grid = e // block

    def body(hr_ref, hc_ref, ea_ref, w1_ref, b1_ref, w2_ref, b2_ref, out_ref):
        w1 = w1_ref[...]
        g = (_dot_t(hr_ref[...], w1[:, :hdim])
             + _dot_t(hc_ref[...], w1[:, hdim:2 * hdim])
             + _dot_t(ea_ref[...], w1[:, 2 * hdim:])
             + b1_ref[...])
        g = _silu(g)
        f = _dot_t(g, w2_ref[...]) + b2_ref[...]
        out_ref[...] = _silu(f)

    return pl.pallas_call(
        body,
        grid=(grid,),
        in_specs=[
            pl.BlockSpec((block, hdim), lambda i: (i, 0)),
            pl.BlockSpec((block, hdim), lambda i: (i, 0)),
            pl.BlockSpec((block, efdim), lambda i: (i, 0)),
            pl.BlockSpec((hdim, 2 * hdim + efdim), lambda i: (0, 0)),
            pl.BlockSpec((1, hdim), lambda i: (0, 0)),
            pl.BlockSpec((hdim, hdim), lambda i: (0, 0)),
            pl.BlockSpec((1, hdim), lambda i: (0, 0)),
        ],
        out_specs=pl.BlockSpec((block, hdim), lambda i: (i, 0)),
        out_shape=jax.ShapeDtypeStruct((e, hdim), jnp.float32),
    )(hr, hc, ea, we1, be1.reshape(1, -1), we2, be2.reshape(1, -1))


def _tc_final(sums2, cnts2, n, half,
              wv11, bv11, wv12, bv12, wv21, bv21, wv22, bv22):
    hi = n - half

    def body(p_ref, c_ref, w11_ref, b11_ref, w12_ref, b12_ref,
             w21_ref, b21_ref, w22_ref, b22_ref, out_ref):
        sums = jnp.concatenate([p_ref[0, :half], p_ref[1, :hi]], axis=0)
        cnt = jnp.concatenate([c_ref[0, :half, :1], c_ref[1, :hi, :1]],
                              axis=0)
        node = sums / jnp.maximum(cnt, 1.0)

        t1 = _silu(_dot_t(node, w11_ref[...]) + b11_ref[...])
        v1 = _dot_t(w12_ref[...], t1) + b12_ref[...]
        t2 = _silu(_dot_t(node, w21_ref[...]) + b21_ref[...])
        v2 = _dot_t(w22_ref[...], t2) + b22_ref[...]

        n1 = jnp.sqrt(jnp.sum(v1 * v1, axis=0, keepdims=True))
        v1n = v1 / jnp.maximum(n1, 1e-12)
        v2 = v2 - jnp.sum(v2 * v1n, axis=0, keepdims=True) * v1n
        n2 = jnp.sqrt(jnp.sum(v2 * v2, axis=0, keepdims=True))
        v2n = v2 / jnp.maximum(n2, 1e-12)
        v3 = jnp.concatenate([
            v1n[1:2] * v2n[2:3] - v1n[2:3] * v2n[1:2],
            v1n[2:3] * v2n[0:1] - v1n[0:1] * v2n[2:3],
            v1n[0:1] * v2n[1:2] - v1n[1:2] * v2n[0:1],
        ], axis=0)
        out_ref[...] = jnp.concatenate([
            v1n[0:1], v2n[0:1], v3[0:1],
            v1n[1:2], v2n[1:2], v3[1:2],
            v1n[2:3], v2n[2:3], v3[2:3],
        ], axis=0)

    return pl.pallas_call(
        body,
        out_shape=jax.ShapeDtypeStruct((9, n), jnp.float32),
    )(sums2, cnts2, wv11, bv11.reshape(1, -1), wv12, bv12.reshape(3, 1),
      wv21, bv21.reshape(1, -1), wv22, bv22.reshape(3, 1))


def kernel(h, x, edge_index, edge_attr, We1, be1, We2, be2,
           Wv11, bv11, Wv12, bv12, Wv21, bv21, Wv22, bv22):
    del x
    n = h.shape[0]
    row = edge_index[0]
    col = edge_index[1]
    hr, hc = _sc_gather(h, row, col)
    ef2 = _tc_edge_mlp(hr, hc, edge_attr, We1, be1, We2, be2)
    (sums2, cnts2), half = _sc_scatter(ef2, row, n)
    out9 = _tc_final(sums2, cnts2, n, half, Wv11, bv11, Wv12, bv12,
                     Wv21, bv21, Wv22, bv22)
    return out9.T.reshape(n, 3, 3)


# speedup vs baseline: 2.7196x; 1.1160x over previous
import functools

import jax
import jax.numpy as jnp
from jax import lax
from jax.experimental import pallas as pl
from jax.experimental.pallas import tpu as pltpu
from jax.experimental.pallas import tpu_sc as plsc

_NC = 2
_NS = 16
_NW = _NC * _NS


def _mesh():
    return plsc.VectorSubcoreMesh(
        core_axis_name="c", subcore_axis_name="s",
        num_cores=_NC, num_subcores=_NS)


def _sc_gather(h, row, col, chunk=80):
    n, hdim = h.shape
    e = row.shape[0]
    per_w = e // _NW
    n_chunks = per_w // chunk

    @functools.partial(
        pl.kernel,
        out_type=(jax.ShapeDtypeStruct((e, hdim), h.dtype),
                  jax.ShapeDtypeStruct((e, hdim), h.dtype)),
        mesh=_mesh(),
        scratch_types=[
            pltpu.VMEM((chunk,), jnp.int32),
            pltpu.VMEM((chunk,), jnp.int32),
            pltpu.VMEM((chunk, hdim), h.dtype),
            pltpu.VMEM((chunk, hdim), h.dtype),
            pltpu.SemaphoreType.DMA,
            pltpu.SemaphoreType.DMA,
        ],
    )
    def k(h_hbm, row_hbm, col_hbm, hr_hbm, hc_hbm,
          idr, idc, bufr, bufc, sem_r, sem_c):
        wid = lax.axis_index("s") * _NC + lax.axis_index("c")
        base = wid * per_w

        def body(j, carry):
            off = base + j * chunk
            pltpu.sync_copy(row_hbm.at[pl.ds(off, chunk)], idr)
            pltpu.sync_copy(col_hbm.at[pl.ds(off, chunk)], idc)
            cr = pltpu.async_copy(h_hbm.at[idr], bufr, sem_r)
            cc = pltpu.async_copy(h_hbm.at[idc], bufc, sem_c)
            cr.wait()
            cc.wait()
            pltpu.sync_copy(bufr, hr_hbm.at[pl.ds(off, chunk)])
            pltpu.sync_copy(bufc, hc_hbm.at[pl.ds(off, chunk)])
            return carry

        lax.fori_loop(0, n_chunks, body, 0)

    return k(h, row, col)


def _sc_scatter(ef, row, n, chunk=80):
    e, hdim = ef.shape
    per_tile = e // _NS
    n_chunks = per_tile // chunk
    half = (-(-n // (2 * 8)) * 8)
    rows_per_tile = -(-(half + 1) // (_NS * 8)) * 8
    acc_rows = rows_per_tile * _NS
    lanes = hdim // 16
    zrows = rows_per_tile // 8

    @functools.partial(
        pl.kernel,
        out_type=(jax.ShapeDtypeStruct((_NC, acc_rows, hdim), jnp.float32),
                  jax.ShapeDtypeStruct((_NC, acc_rows, hdim), jnp.float32)),
        mesh=_mesh(),
        scratch_types=[
            pltpu.VMEM((chunk,), jnp.int32),
            pltpu.VMEM((chunk,), jnp.int32),
            pltpu.VMEM((chunk, hdim), jnp.float32),
            pltpu.VMEM((chunk, hdim), jnp.float32),
            pltpu.VMEM((chunk, hdim), jnp.float32),
            pltpu.VMEM((zrows, hdim), jnp.float32),
            pltpu.VMEM_SHARED((acc_rows, hdim), jnp.float32),
            pltpu.VMEM_SHARED((acc_rows, hdim), jnp.float32),
            pltpu.SemaphoreType.DMA,
            pltpu.SemaphoreType.DMA,
        ],
    )
    def k(ef_hbm, row_hbm, sums_hbm, cnts_hbm,
          idx, idx2, buf0, buf1, ones, zeros, acc_sh, acc1_sh, sem0, sem1):
        cid = lax.axis_index("c")
        sid = lax.axis_index("s")
        base = sid * per_tile
        r0 = sid * rows_per_tile
        lo = cid * half

        def fill(i, carry):
            r = i // lanes
            q = i % lanes
            zeros[r, pl.ds(q * 16, 16)] = jnp.zeros((16,), jnp.float32)
            return carry

        lax.fori_loop(0, zrows * lanes, fill, 0)

        def fill1(i, carry):
            r = i // lanes
            q = i % lanes
            ones[r, pl.ds(q * 16, 16)] = jnp.ones((16,), jnp.float32)
            return carry

        lax.fori_loop(0, chunk * lanes, fill1, 0)

        def localize(j, carry):
            del j, carry
            for q in range(chunk // 16):
                v = idx[pl.ds(q * 16, 16)] - lo
                ok = (v >= 0) & (v < half)
                idx2[pl.ds(q * 16, 16)] = jnp.where(ok, v, half)

        for z in range(rows_per_tile // zrows):
            pltpu.sync_copy(zeros, acc_sh.at[pl.ds(r0 + z * zrows, zrows)])
            pltpu.sync_copy(zeros, acc1_sh.at[pl.ds(r0 + z * zrows, zrows)])

        bufs = (buf0, buf1)
        sems = (sem0, sem1)

        pltpu.async_copy(ef_hbm.at[pl.ds(base, chunk)], bufs[0], sems[0])
        plsc.subcore_barrier()

        def sbody(g, carry):
            for b in range(2):
                j = 2 * g + b

                def start(nj=j + 1, nb=1 - b):
                    pltpu.async_copy(
                        ef_hbm.at[pl.ds(base + nj * chunk, chunk)],
                        bufs[nb], sems[nb])

                pl.when(j + 1 < n_chunks)(start)
                pltpu.sync_copy(row_hbm.at[pl.ds(base + j * chunk, chunk)],
                                idx)
                localize(j, carry)
                pltpu.make_async_copy(
                    ef_hbm.at[pl.ds(base + j * chunk, chunk)],
                    bufs[b], sems[b]).wait()
                pltpu.sync_copy(bufs[b], acc_sh.at[idx2], add=True)
                pltpu.sync_copy(ones, acc1_sh.at[idx2], add=True)
            return carry

        lax.fori_loop(0, n_chunks // 2, sbody, 0)
        plsc.subcore_barrier()
        pltpu.sync_copy(acc_sh.at[pl.ds(r0, rows_per_tile)],
                        sums_hbm.at[cid, pl.ds(r0, rows_per_tile)])
        pltpu.sync_copy(acc1_sh.at[pl.ds(r0, rows_per_tile)],
                        cnts_hbm.at[cid, pl.ds(r0, rows_per_tile)])

    return k(ef, row), half


def _silu(v):
    return v * jax.nn.sigmoid(v)


def _dot_t(a, b):
    return lax.dot_general(a, b, (((1,), (1,)), ((), ())),
                           preferred_element_type=jnp.float32)


def _tc_edge_mlp(hr, hc, ea, we1, be1, we2, be2, block=2000):
    e, hdim = hr.shape
    efdim = ea.shape[1]
    grid = e // block

    def body(hr_ref, hc_ref, ea_ref, w1_ref, b1_ref, w2_ref, b2_ref, out_ref):
        w1 = w1_ref[...]
        g = (_dot_t(hr_ref[...], w1[:, :hdim])
             + _dot_t(hc_ref[...], w1[:, hdim:2 * hdim])
             + _dot_t(ea_ref[...], w1[:, 2 * hdim:])
             + b1_ref[...])
        g = _silu(g)
        f = _dot_t(g, w2_ref[...]) + b2_ref[...]
        out_ref[...] = _silu(f)

    return pl.pallas_call(
        body,
        grid=(grid,),
        in_specs=[
            pl.BlockSpec((block, hdim), lambda i: (i, 0)),
            pl.BlockSpec((block, hdim), lambda i: (i, 0)),
            pl.BlockSpec((block, efdim), lambda i: (i, 0)),
            pl.BlockSpec((hdim, 2 * hdim + efdim), lambda i: (0, 0)),
            pl.BlockSpec((1, hdim), lambda i: (0, 0)),
            pl.BlockSpec((hdim, hdim), lambda i: (0, 0)),
            pl.BlockSpec((1, hdim), lambda i: (0, 0)),
        ],
        out_specs=pl.BlockSpec((block, hdim), lambda i: (i, 0)),
        out_shape=jax.ShapeDtypeStruct((e, hdim), jnp.float32),
    )(hr, hc, ea, we1, be1.reshape(1, -1), we2, be2.reshape(1, -1))


def _tc_final(sums2, cnts2, n, half,
              wv11, bv11, wv12, bv12, wv21, bv21, wv22, bv22):
    hi = n - half

    def body(p_ref, c_ref, w11_ref, b11_ref, w12_ref, b12_ref,
             w21_ref, b21_ref, w22_ref, b22_ref, out_ref):
        sums = jnp.concatenate([p_ref[0, :half], p_ref[1, :hi]], axis=0)
        cnt = jnp.concatenate([c_ref[0, :half, :1], c_ref[1, :hi, :1]],
                              axis=0)
        node = sums / jnp.maximum(cnt, 1.0)

        t1 = _silu(_dot_t(node, w11_ref[...]) + b11_ref[...])
        v1 = _dot_t(w12_ref[...], t1) + b12_ref[...]
        t2 = _silu(_dot_t(node, w21_ref[...]) + b21_ref[...])
        v2 = _dot_t(w22_ref[...], t2) + b22_ref[...]

        n1 = jnp.sqrt(jnp.sum(v1 * v1, axis=0, keepdims=True))
        v1n = v1 / jnp.maximum(n1, 1e-12)
        v2 = v2 - jnp.sum(v2 * v1n, axis=0, keepdims=True) * v1n
        n2 = jnp.sqrt(jnp.sum(v2 * v2, axis=0, keepdims=True))
        v2n = v2 / jnp.maximum(n2, 1e-12)
        v3 = jnp.concatenate([
            v1n[1:2] * v2n[2:3] - v1n[2:3] * v2n[1:2],
            v1n[2:3] * v2n[0:1] - v1n[0:1] * v2n[2:3],
            v1n[0:1] * v2n[1:2] - v1n[1:2] * v2n[0:1],
        ], axis=0)
        out_ref[...] = jnp.concatenate([
            v1n[0:1], v2n[0:1], v3[0:1],
            v1n[1:2], v2n[1:2], v3[1:2],
            v1n[2:3], v2n[2:3], v3[2:3],
        ], axis=0)

    return pl.pallas_call(
        body,
        out_shape=jax.ShapeDtypeStruct((9, n), jnp.float32),
    )(sums2, cnts2, wv11, bv11.reshape(1, -1), wv12, bv12.reshape(3, 1),
      wv21, bv21.reshape(1, -1), wv22, bv22.reshape(3, 1))


def kernel(h, x, edge_index, edge_attr, We1, be1, We2, be2,
           Wv11, bv11, Wv12, bv12, Wv21, bv21, Wv22, bv22):
    del x
    n = h.shape[0]
    row = edge_index[0]
    col = edge_index[1]
    hr, hc = _sc_gather(h, row, col)
    ef2 = _tc_edge_mlp(hr, hc, edge_attr, We1, be1, We2, be2)
    (sums2, cnts2), half = _sc_scatter(ef2, row, n)
    out9 = _tc_final(sums2, cnts2, n, half, Wv11, bv11, Wv12, bv12,
                     Wv21, bv21, Wv22, bv22)
    return out9.T.reshape(n, 3, 3)


# speedup vs baseline: 3.0824x; 1.1334x over previous
import functools

import jax
import jax.numpy as jnp
from jax import lax
from jax.experimental import pallas as pl
from jax.experimental.pallas import tpu as pltpu
from jax.experimental.pallas import tpu_sc as plsc

_NC = 2
_NS = 16
_NW = _NC * _NS


def _mesh():
    return plsc.VectorSubcoreMesh(
        core_axis_name="c", subcore_axis_name="s",
        num_cores=_NC, num_subcores=_NS)


def _sc_gather(h, row, col, chunk=80):
    n, hdim = h.shape
    e = row.shape[0]
    per_w = e // _NW
    n_chunks = per_w // chunk

    @functools.partial(
        pl.kernel,
        out_type=(jax.ShapeDtypeStruct((e, hdim), h.dtype),
                  jax.ShapeDtypeStruct((e, hdim), h.dtype)),
        mesh=_mesh(),
        scratch_types=[
            pltpu.VMEM((per_w,), jnp.int32),
            pltpu.VMEM((per_w,), jnp.int32),
            pltpu.VMEM((chunk, hdim), h.dtype),
            pltpu.VMEM((chunk, hdim), h.dtype),
            pltpu.VMEM((chunk, hdim), h.dtype),
            pltpu.VMEM((chunk, hdim), h.dtype),
            pltpu.SemaphoreType.DMA,
            pltpu.SemaphoreType.DMA,
            pltpu.SemaphoreType.DMA,
            pltpu.SemaphoreType.DMA,
            pltpu.SemaphoreType.DMA,
            pltpu.SemaphoreType.DMA,
            pltpu.SemaphoreType.DMA,
            pltpu.SemaphoreType.DMA,
        ],
    )
    def k(h_hbm, row_hbm, col_hbm, hr_hbm, hc_hbm,
          idrf, idcf, bufr0, bufc0, bufr1, bufc1,
          sgr0, sgc0, sgr1, sgc1, swr0, swc0, swr1, swc1):
        wid = lax.axis_index("s") * _NC + lax.axis_index("c")
        base = wid * per_w
        bufr = (bufr0, bufr1)
        bufc = (bufc0, bufc1)
        sg_r = (sgr0, sgr1)
        sg_c = (sgc0, sgc1)
        sw_r = (swr0, swr1)
        sw_c = (swc0, swc1)

        def start_gather(j, b):
            pltpu.async_copy(h_hbm.at[idrf.at[pl.ds(j * chunk, chunk)]],
                             bufr[b], sg_r[b])
            pltpu.async_copy(h_hbm.at[idcf.at[pl.ds(j * chunk, chunk)]],
                             bufc[b], sg_c[b])

        def wait_gather(j, b):
            pltpu.make_async_copy(
                h_hbm.at[idrf.at[pl.ds(j * chunk, chunk)]],
                bufr[b], sg_r[b]).wait()
            pltpu.make_async_copy(
                h_hbm.at[idcf.at[pl.ds(j * chunk, chunk)]],
                bufc[b], sg_c[b]).wait()

        def start_wb(j, b):
            off = base + j * chunk
            pltpu.async_copy(bufr[b], hr_hbm.at[pl.ds(off, chunk)], sw_r[b])
            pltpu.async_copy(bufc[b], hc_hbm.at[pl.ds(off, chunk)], sw_c[b])

        def wait_wb(j, b):
            off = base + j * chunk
            pltpu.make_async_copy(
                bufr[b], hr_hbm.at[pl.ds(off, chunk)], sw_r[b]).wait()
            pltpu.make_async_copy(
                bufc[b], hc_hbm.at[pl.ds(off, chunk)], sw_c[b]).wait()

        pltpu.sync_copy(row_hbm.at[pl.ds(base, per_w)], idrf)
        pltpu.sync_copy(col_hbm.at[pl.ds(base, per_w)], idcf)
        start_gather(0, 0)
        start_gather(1, 1)
        wait_gather(0, 0)
        start_wb(0, 0)

        def body(g, carry):
            for s in range(2):
                j = 2 * g + 1 + s
                b = (1 + s) % 2
                wait_wb(j - 1, 1 - b)
                start_gather(j + 1, 1 - b)
                wait_gather(j, b)
                start_wb(j, b)
            return carry

        n_pairs = (n_chunks - 2) // 2
        lax.fori_loop(0, n_pairs, body, 0)

        jl = 2 * n_pairs + 1
        wait_wb(jl - 1, 0)
        start_gather(jl + 1, 0)
        wait_gather(jl, 1)
        start_wb(jl, 1)
        wait_gather(jl + 1, 0)
        start_wb(jl + 1, 0)
        wait_wb(jl, 1)
        wait_wb(jl + 1, 0)

    return k(h, row, col)


def _sc_scatter(ef, row, n, chunk=80):
    e, hdim = ef.shape
    per_tile = e // _NS
    n_chunks = per_tile // chunk
    half = (-(-n // (2 * 8)) * 8)
    rows_per_tile = -(-(half + 1) // (_NS * 8)) * 8
    acc_rows = rows_per_tile * _NS
    lanes = hdim // 16
    zrows = rows_per_tile // 8

    @functools.partial(
        pl.kernel,
        out_type=(jax.ShapeDtypeStruct((_NC, acc_rows, hdim), jnp.float32),
                  jax.ShapeDtypeStruct((_NC, acc_rows, hdim), jnp.float32)),
        mesh=_mesh(),
        scratch_types=[
            pltpu.VMEM((chunk,), jnp.int32),
            pltpu.VMEM((chunk,), jnp.int32),
            pltpu.VMEM((chunk, hdim), jnp.float32),
            pltpu.VMEM((chunk, hdim), jnp.float32),
            pltpu.VMEM((chunk, hdim), jnp.float32),
            pltpu.VMEM((zrows, hdim), jnp.float32),
            pltpu.VMEM_SHARED((acc_rows, hdim), jnp.float32),
            pltpu.VMEM_SHARED((acc_rows, hdim), jnp.float32),
            pltpu.SemaphoreType.DMA,
            pltpu.SemaphoreType.DMA,
        ],
    )
    def k(ef_hbm, row_hbm, sums_hbm, cnts_hbm,
          idx, idx2, buf0, buf1, ones, zeros, acc_sh, acc1_sh, sem0, sem1):
        cid = lax.axis_index("c")
        sid = lax.axis_index("s")
        base = sid * per_tile
        r0 = sid * rows_per_tile
        lo = cid * half

        def fill(i, carry):
            r = i // lanes
            q = i % lanes
            zeros[r, pl.ds(q * 16, 16)] = jnp.zeros((16,), jnp.float32)
            return carry

        lax.fori_loop(0, zrows * lanes, fill, 0)

        def fill1(i, carry):
            r = i // lanes
            q = i % lanes
            ones[r, pl.ds(q * 16, 16)] = jnp.ones((16,), jnp.float32)
            return carry

        lax.fori_loop(0, chunk * lanes, fill1, 0)

        def localize(j, carry):
            del j, carry
            for q in range(chunk // 16):
                v = idx[pl.ds(q * 16, 16)] - lo
                ok = (v >= 0) & (v < half)
                idx2[pl.ds(q * 16, 16)] = jnp.where(ok, v, half)

        for z in range(rows_per_tile // zrows):
            pltpu.sync_copy(zeros, acc_sh.at[pl.ds(r0 + z * zrows, zrows)])
            pltpu.sync_copy(zeros, acc1_sh.at[pl.ds(r0 + z * zrows, zrows)])

        bufs = (buf0, buf1)
        sems = (sem0, sem1)

        pltpu.async_copy(ef_hbm.at[pl.ds(base, chunk)], bufs[0], sems[0])
        plsc.subcore_barrier()

        def sbody(g, carry):
            for b in range(2):
                j = 2 * g + b

                def start(nj=j + 1, nb=1 - b):
                    pltpu.async_copy(
                        ef_hbm.at[pl.ds(base + nj * chunk, chunk)],
                        bufs[nb], sems[nb])

                pl.when(j + 1 < n_chunks)(start)
                pltpu.sync_copy(row_hbm.at[pl.ds(base + j * chunk, chunk)],
                                idx)
                localize(j, carry)
                pltpu.make_async_copy(
                    ef_hbm.at[pl.ds(base + j * chunk, chunk)],
                    bufs[b], sems[b]).wait()
                pltpu.sync_copy(bufs[b], acc_sh.at[idx2], add=True)
                pltpu.sync_copy(ones, acc1_sh.at[idx2], add=True)
            return carry

        lax.fori_loop(0, n_chunks // 2, sbody, 0)
        plsc.subcore_barrier()
        pltpu.sync_copy(acc_sh.at[pl.ds(r0, rows_per_tile)],
                        sums_hbm.at[cid, pl.ds(r0, rows_per_tile)])
        pltpu.sync_copy(acc1_sh.at[pl.ds(r0, rows_per_tile)],
                        cnts_hbm.at[cid, pl.ds(r0, rows_per_tile)])

    return k(ef, row), half


def _silu(v):
    return v * jax.nn.sigmoid(v)


def _dot_t(a, b):
    return lax.dot_general(a, b, (((1,), (1,)), ((), ())),
                           preferred_element_type=jnp.float32)


def _tc_edge_mlp(hr, hc, ea, we1, be1, we2, be2, block=2000):
    e, hdim = hr.shape
    efdim = ea.shape[1]
    grid = e // block

    def body(hr_ref, hc_ref, ea_ref, w1_ref, b1_ref, w2_ref, b2_ref, out_ref):
        w1 = w1_ref[...]
        g = (_dot_t(hr_ref[...], w1[:, :hdim])
             + _dot_t(hc_ref[...], w1[:, hdim:2 * hdim])
             + _dot_t(ea_ref[...], w1[:, 2 * hdim:])
             + b1_ref[...])
        g = _silu(g)
        f = _dot_t(g, w2_ref[...]) + b2_ref[...]
        out_ref[...] = _silu(f)

    return pl.pallas_call(
        body,
        grid=(grid,),
        in_specs=[
            pl.BlockSpec((block, hdim), lambda i: (i, 0)),
            pl.BlockSpec((block, hdim), lambda i: (i, 0)),
            pl.BlockSpec((block, efdim), lambda i: (i, 0)),
            pl.BlockSpec((hdim, 2 * hdim + efdim), lambda i: (0, 0)),
            pl.BlockSpec((1, hdim), lambda i: (0, 0)),
            pl.BlockSpec((hdim, hdim), lambda i: (0, 0)),
            pl.BlockSpec((1, hdim), lambda i: (0, 0)),
        ],
        out_specs=pl.BlockSpec((block, hdim), lambda i: (i, 0)),
        out_shape=jax.ShapeDtypeStruct((e, hdim), jnp.float32),
    )(hr, hc, ea, we1, be1.reshape(1, -1), we2, be2.reshape(1, -1))


def _tc_final(sums2, cnts2, n, half,
              wv11, bv11, wv12, bv12, wv21, bv21, wv22, bv22):
    hi = n - half

    def body(p_ref, c_ref, w11_ref, b11_ref, w12_ref, b12_ref,
             w21_ref, b21_ref, w22_ref, b22_ref, out_ref):
        sums = jnp.concatenate([p_ref[0, :half], p_ref[1, :hi]], axis=0)
        cnt = jnp.concatenate([c_ref[0, :half, :1], c_ref[1, :hi, :1]],
                              axis=0)
        node = sums / jnp.maximum(cnt, 1.0)

        t1 = _silu(_dot_t(node, w11_ref[...]) + b11_ref[...])
        v1 = _dot_t(w12_ref[...], t1) + b12_ref[...]
        t2 = _silu(_dot_t(node, w21_ref[...]) + b21_ref[...])
        v2 = _dot_t(w22_ref[...], t2) + b22_ref[...]

        n1 = jnp.sqrt(jnp.sum(v1 * v1, axis=0, keepdims=True))
        v1n = v1 / jnp.maximum(n1, 1e-12)
        v2 = v2 - jnp.sum(v2 * v1n, axis=0, keepdims=True) * v1n
        n2 = jnp.sqrt(jnp.sum(v2 * v2, axis=0, keepdims=True))
        v2n = v2 / jnp.maximum(n2, 1e-12)
        v3 = jnp.concatenate([
            v1n[1:2] * v2n[2:3] - v1n[2:3] * v2n[1:2],
            v1n[2:3] * v2n[0:1] - v1n[0:1] * v2n[2:3],
            v1n[0:1] * v2n[1:2] - v1n[1:2] * v2n[0:1],
        ], axis=0)
        out_ref[...] = jnp.concatenate([
            v1n[0:1], v2n[0:1], v3[0:1],
            v1n[1:2], v2n[1:2], v3[1:2],
            v1n[2:3], v2n[2:3], v3[2:3],
        ], axis=0)

    return pl.pallas_call(
        body,
        out_shape=jax.ShapeDtypeStruct((9, n), jnp.float32),
    )(sums2, cnts2, wv11, bv11.reshape(1, -1), wv12, bv12.reshape(3, 1),
      wv21, bv21.reshape(1, -1), wv22, bv22.reshape(3, 1))


def kernel(h, x, edge_index, edge_attr, We1, be1, We2, be2,
           Wv11, bv11, Wv12, bv12, Wv21, bv21, Wv22, bv22):
    del x
    n = h.shape[0]
    row = edge_index[0]
    col = edge_index[1]
    hr, hc = _sc_gather(h, row, col)
    ef2 = _tc_edge_mlp(hr, hc, edge_attr, We1, be1, We2, be2)
    (sums2, cnts2), half = _sc_scatter(ef2, row, n)
    out9 = _tc_final(sums2, cnts2, n, half, Wv11, bv11, Wv12, bv12,
                     Wv21, bv21, Wv22, bv22)
    return out9.T.reshape(n, 3, 3)


# speedup vs baseline: 3.5623x; 1.1557x over previous
import functools

import jax
import jax.numpy as jnp
from jax import lax
from jax.experimental import pallas as pl
from jax.experimental.pallas import tpu as pltpu
from jax.experimental.pallas import tpu_sc as plsc

_NC = 2
_NS = 16
_NW = _NC * _NS


def _mesh():
    return plsc.VectorSubcoreMesh(
        core_axis_name="c", subcore_axis_name="s",
        num_cores=_NC, num_subcores=_NS)


def _sc_gather(h, row, col, ne, off, chunk=40):
    n, hdim = h.shape
    per_w = ne // _NW
    n_chunks = per_w // chunk

    @functools.partial(
        pl.kernel,
        out_type=(jax.ShapeDtypeStruct((ne, hdim), h.dtype),
                  jax.ShapeDtypeStruct((ne, hdim), h.dtype)),
        mesh=_mesh(),
        scratch_types=[
            pltpu.VMEM((per_w,), jnp.int32),
            pltpu.VMEM((per_w,), jnp.int32),
            pltpu.VMEM((chunk, hdim), h.dtype),
            pltpu.VMEM((chunk, hdim), h.dtype),
            pltpu.VMEM((chunk, hdim), h.dtype),
            pltpu.VMEM((chunk, hdim), h.dtype),
            pltpu.SemaphoreType.DMA,
            pltpu.SemaphoreType.DMA,
            pltpu.SemaphoreType.DMA,
            pltpu.SemaphoreType.DMA,
            pltpu.SemaphoreType.DMA,
            pltpu.SemaphoreType.DMA,
            pltpu.SemaphoreType.DMA,
            pltpu.SemaphoreType.DMA,
        ],
    )
    def k(h_hbm, row_hbm, col_hbm, hr_hbm, hc_hbm,
          idrf, idcf, bufr0, bufc0, bufr1, bufc1,
          sgr0, sgc0, sgr1, sgc1, swr0, swc0, swr1, swc1):
        wid = lax.axis_index("s") * _NC + lax.axis_index("c")
        base = wid * per_w
        bufr = (bufr0, bufr1)
        bufc = (bufc0, bufc1)
        sg_r = (sgr0, sgr1)
        sg_c = (sgc0, sgc1)
        sw_r = (swr0, swr1)
        sw_c = (swc0, swc1)

        def start_gather(j, b):
            pltpu.async_copy(h_hbm.at[idrf.at[pl.ds(j * chunk, chunk)]],
                             bufr[b], sg_r[b])
            pltpu.async_copy(h_hbm.at[idcf.at[pl.ds(j * chunk, chunk)]],
                             bufc[b], sg_c[b])

        def wait_gather(j, b):
            pltpu.make_async_copy(
                h_hbm.at[idrf.at[pl.ds(j * chunk, chunk)]],
                bufr[b], sg_r[b]).wait()
            pltpu.make_async_copy(
                h_hbm.at[idcf.at[pl.ds(j * chunk, chunk)]],
                bufc[b], sg_c[b]).wait()

        def start_wb(j, b):
            off = base + j * chunk
            pltpu.async_copy(bufr[b], hr_hbm.at[pl.ds(off, chunk)], sw_r[b])
            pltpu.async_copy(bufc[b], hc_hbm.at[pl.ds(off, chunk)], sw_c[b])

        def wait_wb(j, b):
            off = base + j * chunk
            pltpu.make_async_copy(
                bufr[b], hr_hbm.at[pl.ds(off, chunk)], sw_r[b]).wait()
            pltpu.make_async_copy(
                bufc[b], hc_hbm.at[pl.ds(off, chunk)], sw_c[b]).wait()

        pltpu.sync_copy(row_hbm.at[pl.ds(off + base, per_w)], idrf)
        pltpu.sync_copy(col_hbm.at[pl.ds(off + base, per_w)], idcf)
        start_gather(0, 0)
        start_gather(1, 1)
        wait_gather(0, 0)
        start_wb(0, 0)

        def body(g, carry):
            for s in range(2):
                j = 2 * g + 1 + s
                b = (1 + s) % 2
                wait_wb(j - 1, 1 - b)
                start_gather(j + 1, 1 - b)
                wait_gather(j, b)
                start_wb(j, b)
            return carry

        n_pairs = (n_chunks - 2) // 2
        lax.fori_loop(0, n_pairs, body, 0)

        jl = 2 * n_pairs + 1
        wait_wb(jl - 1, 0)
        start_gather(jl + 1, 0)
        wait_gather(jl, 1)
        start_wb(jl, 1)
        wait_gather(jl + 1, 0)
        start_wb(jl + 1, 0)
        wait_wb(jl, 1)
        wait_wb(jl + 1, 0)

    return k(h, row, col)


def _sc_scatter(ef, row, n, off, chunk=80):
    e, hdim = ef.shape
    per_tile = e // _NS
    n_chunks = per_tile // chunk
    half = (-(-n // (2 * 8)) * 8)
    rows_per_tile = -(-(half + 1) // (_NS * 8)) * 8
    acc_rows = rows_per_tile * _NS
    lanes = hdim // 16
    zrows = rows_per_tile // 8

    @functools.partial(
        pl.kernel,
        out_type=(jax.ShapeDtypeStruct((_NC, acc_rows, hdim), jnp.float32),
                  jax.ShapeDtypeStruct((_NC, acc_rows, hdim), jnp.float32)),
        mesh=_mesh(),
        scratch_types=[
            pltpu.VMEM((chunk,), jnp.int32),
            pltpu.VMEM((chunk,), jnp.int32),
            pltpu.VMEM((chunk, hdim), jnp.float32),
            pltpu.VMEM((chunk, hdim), jnp.float32),
            pltpu.VMEM((chunk, hdim), jnp.float32),
            pltpu.VMEM((zrows, hdim), jnp.float32),
            pltpu.VMEM_SHARED((acc_rows, hdim), jnp.float32),
            pltpu.VMEM_SHARED((acc_rows, hdim), jnp.float32),
            pltpu.SemaphoreType.DMA,
            pltpu.SemaphoreType.DMA,
        ],
    )
    def k(ef_hbm, row_hbm, sums_hbm, cnts_hbm,
          idx, idx2, buf0, buf1, ones, zeros, acc_sh, acc1_sh, sem0, sem1):
        cid = lax.axis_index("c")
        sid = lax.axis_index("s")
        base = sid * per_tile
        r0 = sid * rows_per_tile
        lo = cid * half

        def fill(i, carry):
            r = i // lanes
            q = i % lanes
            zeros[r, pl.ds(q * 16, 16)] = jnp.zeros((16,), jnp.float32)
            return carry

        lax.fori_loop(0, zrows * lanes, fill, 0)

        def fill1(i, carry):
            r = i // lanes
            q = i % lanes
            ones[r, pl.ds(q * 16, 16)] = jnp.ones((16,), jnp.float32)
            return carry

        lax.fori_loop(0, chunk * lanes, fill1, 0)

        def localize(j, carry):
            del j, carry
            for q in range(chunk // 16):
                v = idx[pl.ds(q * 16, 16)] - lo
                ok = (v >= 0) & (v < half)
                idx2[pl.ds(q * 16, 16)] = jnp.where(ok, v, half)

        for z in range(rows_per_tile // zrows):
            pltpu.sync_copy(zeros, acc_sh.at[pl.ds(r0 + z * zrows, zrows)])
            pltpu.sync_copy(zeros, acc1_sh.at[pl.ds(r0 + z * zrows, zrows)])

        bufs = (buf0, buf1)
        sems = (sem0, sem1)

        pltpu.async_copy(ef_hbm.at[pl.ds(base, chunk)], bufs[0], sems[0])
        plsc.subcore_barrier()

        def step(j, b):
            def start(nj=None, nb=1 - b):
                pltpu.async_copy(
                    ef_hbm.at[pl.ds(base + nj * chunk, chunk)],
                    bufs[nb], sems[nb])

            pl.when(j + 1 < n_chunks)(
                functools.partial(start, nj=j + 1))
            pltpu.sync_copy(
                row_hbm.at[pl.ds(off + base + j * chunk, chunk)], idx)
            localize(j, 0)
            pltpu.make_async_copy(
                ef_hbm.at[pl.ds(base + j * chunk, chunk)],
                bufs[b], sems[b]).wait()
            pltpu.sync_copy(bufs[b], acc_sh.at[idx2], add=True)
            pltpu.sync_copy(ones, acc1_sh.at[idx2], add=True)

        def sbody(g, carry):
            for b in range(2):
                step(2 * g + b, b)
            return carry

        lax.fori_loop(0, n_chunks // 2, sbody, 0)
        if n_chunks % 2:
            step(n_chunks - 1, 0)
        plsc.subcore_barrier()
        pltpu.sync_copy(acc_sh.at[pl.ds(r0, rows_per_tile)],
                        sums_hbm.at[cid, pl.ds(r0, rows_per_tile)])
        pltpu.sync_copy(acc1_sh.at[pl.ds(r0, rows_per_tile)],
                        cnts_hbm.at[cid, pl.ds(r0, rows_per_tile)])

    return k(ef, row), half


def _silu(v):
    return v * jax.nn.sigmoid(v)


def _dot_t(a, b):
    return lax.dot_general(a, b, (((1,), (1,)), ((), ())),
                           preferred_element_type=jnp.float32)


def _tc_edge_mlp(hr, hc, ea, off, we1, be1, we2, be2, block=2000):
    e, hdim = hr.shape
    efdim = ea.shape[1]
    grid = e // block
    ob = off // block

    def body(hr_ref, hc_ref, ea_ref, w1_ref, b1_ref, w2_ref, b2_ref, out_ref):
        w1 = w1_ref[...]
        g = (_dot_t(hr_ref[...], w1[:, :hdim])
             + _dot_t(hc_ref[...], w1[:, hdim:2 * hdim])
             + _dot_t(ea_ref[...], w1[:, 2 * hdim:])
             + b1_ref[...])
        g = _silu(g)
        f = _dot_t(g, w2_ref[...]) + b2_ref[...]
        out_ref[...] = _silu(f)

    return pl.pallas_call(
        body,
        grid=(grid,),
        in_specs=[
            pl.BlockSpec((block, hdim), lambda i: (i, 0)),
            pl.BlockSpec((block, hdim), lambda i: (i, 0)),
            pl.BlockSpec((block, efdim), lambda i: (i + ob, 0)),
            pl.BlockSpec((hdim, 2 * hdim + efdim), lambda i: (0, 0)),
            pl.BlockSpec((1, hdim), lambda i: (0, 0)),
            pl.BlockSpec((hdim, hdim), lambda i: (0, 0)),
            pl.BlockSpec((1, hdim), lambda i: (0, 0)),
        ],
        out_specs=pl.BlockSpec((block, hdim), lambda i: (i, 0)),
        out_shape=jax.ShapeDtypeStruct((e, hdim), jnp.float32),
    )(hr, hc, ea, we1, be1.reshape(1, -1), we2, be2.reshape(1, -1))


def _tc_final(sumsA, cntsA, sumsB, cntsB, n, half,
              wv11, bv11, wv12, bv12, wv21, bv21, wv22, bv22):
    hi = n - half

    def body(pa_ref, ca_ref, pb_ref, cb_ref,
             w11_ref, b11_ref, w12_ref, b12_ref,
             w21_ref, b21_ref, w22_ref, b22_ref, out_ref):
        sums = jnp.concatenate(
            [pa_ref[0, :half] + pb_ref[0, :half],
             pa_ref[1, :hi] + pb_ref[1, :hi]], axis=0)
        cnt = jnp.concatenate(
            [ca_ref[0, :half, :1] + cb_ref[0, :half, :1],
             ca_ref[1, :hi, :1] + cb_ref[1, :hi, :1]], axis=0)
        node = sums / jnp.maximum(cnt, 1.0)

        t1 = _silu(_dot_t(node, w11_ref[...]) + b11_ref[...])
        v1 = _dot_t(w12_ref[...], t1) + b12_ref[...]
        t2 = _silu(_dot_t(node, w21_ref[...]) + b21_ref[...])
        v2 = _dot_t(w22_ref[...], t2) + b22_ref[...]

        n1 = jnp.sqrt(jnp.sum(v1 * v1, axis=0, keepdims=True))
        v1n = v1 / jnp.maximum(n1, 1e-12)
        v2 = v2 - jnp.sum(v2 * v1n, axis=0, keepdims=True) * v1n
        n2 = jnp.sqrt(jnp.sum(v2 * v2, axis=0, keepdims=True))
        v2n = v2 / jnp.maximum(n2, 1e-12)
        v3 = jnp.concatenate([
            v1n[1:2] * v2n[2:3] - v1n[2:3] * v2n[1:2],
            v1n[2:3] * v2n[0:1] - v1n[0:1] * v2n[2:3],
            v1n[0:1] * v2n[1:2] - v1n[1:2] * v2n[0:1],
        ], axis=0)
        out_ref[...] = jnp.concatenate([
            v1n[0:1], v2n[0:1], v3[0:1],
            v1n[1:2], v2n[1:2], v3[1:2],
            v1n[2:3], v2n[2:3], v3[2:3],
        ], axis=0)

    return pl.pallas_call(
        body,
        out_shape=jax.ShapeDtypeStruct((9, n), jnp.float32),
    )(sumsA, cntsA, sumsB, cntsB,
      wv11, bv11.reshape(1, -1), wv12, bv12.reshape(3, 1),
      wv21, bv21.reshape(1, -1), wv22, bv22.reshape(3, 1))


def kernel(h, x, edge_index, edge_attr, We1, be1, We2, be2,
           Wv11, bv11, Wv12, bv12, Wv21, bv21, Wv22, bv22):
    del x
    n = h.shape[0]
    e = edge_index.shape[1]
    eh = e // 2
    row = edge_index[0]
    col = edge_index[1]
    hrA, hcA = _sc_gather(h, row, col, eh, 0)
    hrB, hcB = _sc_gather(h, row, col, eh, eh)
    efA = _tc_edge_mlp(hrA, hcA, edge_attr, 0, We1, be1, We2, be2)
    efB = _tc_edge_mlp(hrB, hcB, edge_attr, eh, We1, be1, We2, be2)
    (sumsA, cntsA), half = _sc_scatter(efA, row, n, 0)
    (sumsB, cntsB), _ = _sc_scatter(efB, row, n, eh)
    out9 = _tc_final(sumsA, cntsA, sumsB, cntsB, n, half,
                     Wv11, bv11, Wv12, bv12, Wv21, bv21, Wv22, bv22)
    return out9.T.reshape(n, 3, 3)


# speedup vs baseline: 3.8676x; 1.0857x over previous
import functools

import jax
import jax.numpy as jnp
from jax import lax
from jax.experimental import pallas as pl
from jax.experimental.pallas import tpu as pltpu
from jax.experimental.pallas import tpu_sc as plsc

_NC = 2
_NS = 16
_NW = _NC * _NS


def _mesh():
    return plsc.VectorSubcoreMesh(
        core_axis_name="c", subcore_axis_name="s",
        num_cores=_NC, num_subcores=_NS)


def _sc_gather(h, row, col, ne, off, chunk=80):
    n, hdim = h.shape
    per_w = ne // _NW
    n_chunks = per_w // chunk

    @functools.partial(
        pl.kernel,
        out_type=(jax.ShapeDtypeStruct((ne, hdim), h.dtype),
                  jax.ShapeDtypeStruct((ne, hdim), h.dtype)),
        mesh=_mesh(),
        scratch_types=[
            pltpu.VMEM((per_w,), jnp.int32),
            pltpu.VMEM((per_w,), jnp.int32),
            pltpu.VMEM((chunk, hdim), h.dtype),
            pltpu.VMEM((chunk, hdim), h.dtype),
            pltpu.VMEM((chunk, hdim), h.dtype),
            pltpu.VMEM((chunk, hdim), h.dtype),
            pltpu.SemaphoreType.DMA,
            pltpu.SemaphoreType.DMA,
            pltpu.SemaphoreType.DMA,
            pltpu.SemaphoreType.DMA,
            pltpu.SemaphoreType.DMA,
            pltpu.SemaphoreType.DMA,
            pltpu.SemaphoreType.DMA,
            pltpu.SemaphoreType.DMA,
        ],
    )
    def k(h_hbm, row_hbm, col_hbm, hr_hbm, hc_hbm,
          idrf, idcf, bufr0, bufc0, bufr1, bufc1,
          sgr0, sgc0, sgr1, sgc1, swr0, swc0, swr1, swc1):
        wid = lax.axis_index("s") * _NC + lax.axis_index("c")
        base = wid * per_w
        bufr = (bufr0, bufr1)
        bufc = (bufc0, bufc1)
        sg_r = (sgr0, sgr1)
        sg_c = (sgc0, sgc1)
        sw_r = (swr0, swr1)
        sw_c = (swc0, swc1)

        def start_gather(j, b):
            pltpu.async_copy(h_hbm.at[idrf.at[pl.ds(j * chunk, chunk)]],
                             bufr[b], sg_r[b])
            pltpu.async_copy(h_hbm.at[idcf.at[pl.ds(j * chunk, chunk)]],
                             bufc[b], sg_c[b])

        def wait_gather(j, b):
            pltpu.make_async_copy(
                h_hbm.at[idrf.at[pl.ds(j * chunk, chunk)]],
                bufr[b], sg_r[b]).wait()
            pltpu.make_async_copy(
                h_hbm.at[idcf.at[pl.ds(j * chunk, chunk)]],
                bufc[b], sg_c[b]).wait()

        def start_wb(j, b):
            off = base + j * chunk
            pltpu.async_copy(bufr[b], hr_hbm.at[pl.ds(off, chunk)], sw_r[b])
            pltpu.async_copy(bufc[b], hc_hbm.at[pl.ds(off, chunk)], sw_c[b])

        def wait_wb(j, b):
            off = base + j * chunk
            pltpu.make_async_copy(
                bufr[b], hr_hbm.at[pl.ds(off, chunk)], sw_r[b]).wait()
            pltpu.make_async_copy(
                bufc[b], hc_hbm.at[pl.ds(off, chunk)], sw_c[b]).wait()

        pltpu.sync_copy(row_hbm.at[pl.ds(off + base, per_w)], idrf)
        pltpu.sync_copy(col_hbm.at[pl.ds(off + base, per_w)], idcf)
        start_gather(0, 0)
        start_gather(1, 1)
        wait_gather(0, 0)
        start_wb(0, 0)

        def body(g, carry):
            for s in range(2):
                j = 2 * g + 1 + s
                b = (1 + s) % 2
                wait_wb(j - 1, 1 - b)
                start_gather(j + 1, 1 - b)
                wait_gather(j, b)
                start_wb(j, b)
            return carry

        n_pairs = (n_chunks - 2) // 2
        lax.fori_loop(0, n_pairs, body, 0)

        jl = 2 * n_pairs + 1
        if n_chunks % 2:
            wait_wb(jl - 1, 0)
            start_gather(jl + 1, 0)
            wait_gather(jl, 1)
            start_wb(jl, 1)
            wait_gather(jl + 1, 0)
            start_wb(jl + 1, 0)
            wait_wb(jl, 1)
            wait_wb(jl + 1, 0)
        else:
            wait_gather(jl, 1)
            start_wb(jl, 1)
            wait_wb(jl - 1, 0)
            wait_wb(jl, 1)

    return k(h, row, col)


def _sc_scatter(ef, row, n, off, chunk=80):
    e, hdim = ef.shape
    per_tile = e // _NS
    n_chunks = per_tile // chunk
    half = (-(-n // (2 * 8)) * 8)
    rows_per_tile = -(-(half + 1) // (_NS * 8)) * 8
    acc_rows = rows_per_tile * _NS
    lanes = hdim // 16
    zrows = rows_per_tile // 8

    @functools.partial(
        pl.kernel,
        out_type=(jax.ShapeDtypeStruct((_NC, acc_rows, hdim), jnp.float32),
                  jax.ShapeDtypeStruct((_NC, acc_rows, hdim), jnp.float32)),
        mesh=_mesh(),
        scratch_types=[
            pltpu.VMEM((chunk,), jnp.int32),
            pltpu.VMEM((chunk,), jnp.int32),
            pltpu.VMEM((chunk, hdim), jnp.float32),
            pltpu.VMEM((chunk, hdim), jnp.float32),
            pltpu.VMEM((chunk, hdim), jnp.float32),
            pltpu.VMEM((zrows, hdim), jnp.float32),
            pltpu.VMEM_SHARED((acc_rows, hdim), jnp.float32),
            pltpu.VMEM_SHARED((acc_rows, hdim), jnp.float32),
            pltpu.SemaphoreType.DMA,
            pltpu.SemaphoreType.DMA,
        ],
    )
    def k(ef_hbm, row_hbm, sums_hbm, cnts_hbm,
          idx, idx2, buf0, buf1, ones, zeros, acc_sh, acc1_sh, sem0, sem1):
        cid = lax.axis_index("c")
        sid = lax.axis_index("s")
        base = sid * per_tile
        r0 = sid * rows_per_tile
        lo = cid * half

        def fill(i, carry):
            r = i // lanes
            q = i % lanes
            zeros[r, pl.ds(q * 16, 16)] = jnp.zeros((16,), jnp.float32)
            return carry

        lax.fori_loop(0, zrows * lanes, fill, 0)

        def fill1(i, carry):
            r = i // lanes
            q = i % lanes
            ones[r, pl.ds(q * 16, 16)] = jnp.ones((16,), jnp.float32)
            return carry

        lax.fori_loop(0, chunk * lanes, fill1, 0)

        def localize(j, carry):
            del j, carry
            for q in range(chunk // 16):
                v = idx[pl.ds(q * 16, 16)] - lo
                ok = (v >= 0) & (v < half)
                idx2[pl.ds(q * 16, 16)] = jnp.where(ok, v, half + sid)

        for z in range(rows_per_tile // zrows):
            pltpu.sync_copy(zeros, acc_sh.at[pl.ds(r0 + z * zrows, zrows)])
            pltpu.sync_copy(zeros, acc1_sh.at[pl.ds(r0 + z * zrows, zrows)])

        bufs = (buf0, buf1)
        sems = (sem0, sem1)

        pltpu.async_copy(ef_hbm.at[pl.ds(base, chunk)], bufs[0], sems[0])
        plsc.subcore_barrier()

        def step(j, b):
            def start(nj=None, nb=1 - b):
                pltpu.async_copy(
                    ef_hbm.at[pl.ds(base + nj * chunk, chunk)],
                    bufs[nb], sems[nb])

            pl.when(j + 1 < n_chunks)(
                functools.partial(start, nj=j + 1))
            pltpu.sync_copy(
                row_hbm.at[pl.ds(off + base + j * chunk, chunk)], idx)
            localize(j, 0)
            pltpu.make_async_copy(
                ef_hbm.at[pl.ds(base + j * chunk, chunk)],
                bufs[b], sems[b]).wait()
            pltpu.sync_copy(bufs[b], acc_sh.at[idx2], add=True)
            pltpu.sync_copy(ones, acc1_sh.at[idx2], add=True)

        def sbody(g, carry):
            for b in range(2):
                step(2 * g + b, b)
            return carry

        lax.fori_loop(0, n_chunks // 2, sbody, 0)
        if n_chunks % 2:
            step(n_chunks - 1, 0)
        plsc.subcore_barrier()
        pltpu.sync_copy(acc_sh.at[pl.ds(r0, rows_per_tile)],
                        sums_hbm.at[cid, pl.ds(r0, rows_per_tile)])
        pltpu.sync_copy(acc1_sh.at[pl.ds(r0, rows_per_tile)],
                        cnts_hbm.at[cid, pl.ds(r0, rows_per_tile)])

    return k(ef, row), half


def _silu(v):
    return v * jax.nn.sigmoid(v)


def _dot_t(a, b):
    return lax.dot_general(a, b, (((1,), (1,)), ((), ())),
                           preferred_element_type=jnp.float32)


def _tc_edge_mlp(hr, hc, ea, off, we1, be1, we2, be2, block=1600):
    e, hdim = hr.shape
    efdim = ea.shape[1]
    grid = e // block
    ob = off // block

    def body(hr_ref, hc_ref, ea_ref, w1_ref, b1_ref, w2_ref, b2_ref, out_ref):
        w1 = w1_ref[...]
        g = (_dot_t(hr_ref[...], w1[:, :hdim])
             + _dot_t(hc_ref[...], w1[:, hdim:2 * hdim])
             + _dot_t(ea_ref[...], w1[:, 2 * hdim:])
             + b1_ref[...])
        g = _silu(g)
        f = _dot_t(g, w2_ref[...]) + b2_ref[...]
        out_ref[...] = _silu(f)

    return pl.pallas_call(
        body,
        grid=(grid,),
        in_specs=[
            pl.BlockSpec((block, hdim), lambda i: (i, 0)),
            pl.BlockSpec((block, hdim), lambda i: (i, 0)),
            pl.BlockSpec((block, efdim), lambda i: (i + ob, 0)),
            pl.BlockSpec((hdim, 2 * hdim + efdim), lambda i: (0, 0)),
            pl.BlockSpec((1, hdim), lambda i: (0, 0)),
            pl.BlockSpec((hdim, hdim), lambda i: (0, 0)),
            pl.BlockSpec((1, hdim), lambda i: (0, 0)),
        ],
        out_specs=pl.BlockSpec((block, hdim), lambda i: (i, 0)),
        out_shape=jax.ShapeDtypeStruct((e, hdim), jnp.float32),
    )(hr, hc, ea, we1, be1.reshape(1, -1), we2, be2.reshape(1, -1))


def _tc_final(sumsA, cntsA, sumsB, cntsB, n, half,
              wv11, bv11, wv12, bv12, wv21, bv21, wv22, bv22):
    hi = n - half

    def body(pa_ref, ca_ref, pb_ref, cb_ref,
             w11_ref, b11_ref, w12_ref, b12_ref,
             w21_ref, b21_ref, w22_ref, b22_ref, out_ref):
        sums = jnp.concatenate(
            [pa_ref[0, :half] + pb_ref[0, :half],
             pa_ref[1, :hi] + pb_ref[1, :hi]], axis=0)
        cnt = jnp.concatenate(
            [ca_ref[0, :half, :1] + cb_ref[0, :half, :1],
             ca_ref[1, :hi, :1] + cb_ref[1, :hi, :1]], axis=0)
        node = sums / jnp.maximum(cnt, 1.0)

        t1 = _silu(_dot_t(node, w11_ref[...]) + b11_ref[...])
        v1 = _dot_t(w12_ref[...], t1) + b12_ref[...]
        t2 = _silu(_dot_t(node, w21_ref[...]) + b21_ref[...])
        v2 = _dot_t(w22_ref[...], t2) + b22_ref[...]

        n1 = jnp.sqrt(jnp.sum(v1 * v1, axis=0, keepdims=True))
        v1n = v1 / jnp.maximum(n1, 1e-12)
        v2 = v2 - jnp.sum(v2 * v1n, axis=0, keepdims=True) * v1n
        n2 = jnp.sqrt(jnp.sum(v2 * v2, axis=0, keepdims=True))
        v2n = v2 / jnp.maximum(n2, 1e-12)
        v3 = jnp.concatenate([
            v1n[1:2] * v2n[2:3] - v1n[2:3] * v2n[1:2],
            v1n[2:3] * v2n[0:1] - v1n[0:1] * v2n[2:3],
            v1n[0:1] * v2n[1:2] - v1n[1:2] * v2n[0:1],
        ], axis=0)
        out_ref[...] = jnp.concatenate([
            v1n[0:1], v2n[0:1], v3[0:1],
            v1n[1:2], v2n[1:2], v3[1:2],
            v1n[2:3], v2n[2:3], v3[2:3],
        ], axis=0)

    return pl.pallas_call(
        body,
        out_shape=jax.ShapeDtypeStruct((9, n), jnp.float32),
    )(sumsA, cntsA, sumsB, cntsB,
      wv11, bv11.reshape(1, -1), wv12, bv12.reshape(3, 1),
      wv21, bv21.reshape(1, -1), wv22, bv22.reshape(3, 1))


def kernel(h, x, edge_index, edge_attr, We1, be1, We2, be2,
           Wv11, bv11, Wv12, bv12, Wv21, bv21, Wv22, bv22):
    del x
    n = h.shape[0]
    e = edge_index.shape[1]
    ea_sz = (e * 13 // 25) // 2560 * 2560
    row = edge_index[0]
    col = edge_index[1]
    hrA, hcA = _sc_gather(h, row, col, ea_sz, 0)
    hrB, hcB = _sc_gather(h, row, col, e - ea_sz, ea_sz)
    efA = _tc_edge_mlp(hrA, hcA, edge_attr, 0, We1, be1, We2, be2)
    efB = _tc_edge_mlp(hrB, hcB, edge_attr, ea_sz, We1, be1, We2, be2)
    (sumsA, cntsA), half = _sc_scatter(efA, row, n, 0)
    (sumsB, cntsB), _ = _sc_scatter(efB, row, n, ea_sz)
    out9 = _tc_final(sumsA, cntsA, sumsB, cntsB, n, half,
                     Wv11, bv11, Wv12, bv12, Wv21, bv21, Wv22, bv22)
    return out9.T.reshape(n, 3, 3)
